# Initial kernel scaffold; baseline (speedup 1.0000x reference)
import jax
import jax.numpy as jnp
from jax.experimental import pallas as pl


def kernel(x, edge_index, edge_attr, W, b):
    raise NotImplementedError("write your pallas kernel here")


# speedup vs baseline: 4.1627x; 4.1627x over previous
import functools

import jax
import jax.numpy as jnp
from jax import lax
from jax.experimental import pallas as pl
from jax.experimental.pallas import tpu as pltpu
from jax.experimental.pallas import tpu_sc as plsc


def _sc_segment_sums(x, row, col, edge_attr):
  n, d = x.shape
  ne = row.shape[0]
  da = edge_attr.shape[1]
  info = plsc.get_sparse_core_info()
  nc, ns = info.num_cores, info.num_subcores
  nw = nc * ns
  per_w = ne // nw
  chunk = 80
  n_chunks = per_w // chunk
  n_pad = -(-n // (8 * ns)) * (8 * ns)
  rows_per_tile = n_pad // ns

  zs = jnp.zeros((n_pad, d), jnp.float32)
  ze = jnp.zeros((n_pad, da), jnp.float32)
  ones = jnp.ones((chunk, da), jnp.float32)

  mesh = plsc.VectorSubcoreMesh(core_axis_name="c", subcore_axis_name="s")

  @functools.partial(
      pl.kernel,
      out_type=(
          jax.ShapeDtypeStruct((nc, n_pad, d), jnp.float32),
          jax.ShapeDtypeStruct((nc, n_pad, da), jnp.float32),
          jax.ShapeDtypeStruct((nc, n_pad, da), jnp.float32),
      ),
      mesh=mesh,
      compiler_params=pltpu.CompilerParams(use_tc_tiling_on_sc=False),
      scratch_types=[
          pltpu.VMEM_SHARED((n_pad, d), jnp.float32),
          pltpu.VMEM_SHARED((n_pad, da), jnp.float32),
          pltpu.VMEM_SHARED((n_pad, da), jnp.float32),
          pltpu.VMEM((chunk,), jnp.int32),
          pltpu.VMEM((chunk,), jnp.int32),
          pltpu.VMEM((chunk, d), jnp.float32),
          pltpu.VMEM((chunk, da), jnp.float32),
          pltpu.VMEM((chunk, da), jnp.float32),
          pltpu.SemaphoreType.DMA,
      ],
  )
  def k(x_hbm, row_hbm, col_hbm, attr_hbm, zs_hbm, ze_hbm, ones_hbm,
        s_out, e_out, c_out,
        sh_s, sh_e, sh_c, col_v, row_v, xbuf, abuf, ones_v, sem):
    cid = lax.axis_index("c")
    sid = lax.axis_index("s")
    wid = sid * nc + cid
    r0 = sid * rows_per_tile
    pltpu.sync_copy(zs_hbm.at[pl.ds(r0, rows_per_tile)],
                    sh_s.at[pl.ds(r0, rows_per_tile)])
    pltpu.sync_copy(ze_hbm.at[pl.ds(r0, rows_per_tile)],
                    sh_e.at[pl.ds(r0, rows_per_tile)])
    pltpu.sync_copy(ze_hbm.at[pl.ds(r0, rows_per_tile)],
                    sh_c.at[pl.ds(r0, rows_per_tile)])
    pltpu.sync_copy(ones_hbm, ones_v)
    plsc.subcore_barrier()

    def body(j, carry):
      base = wid * per_w + j * chunk
      pltpu.sync_copy(col_hbm.at[pl.ds(base, chunk)], col_v)
      pltpu.sync_copy(row_hbm.at[pl.ds(base, chunk)], row_v)
      pltpu.async_copy(x_hbm.at[col_v], xbuf, sem).wait()
      pltpu.sync_copy(xbuf, sh_s.at[row_v], add=True)
      pltpu.sync_copy(attr_hbm.at[pl.ds(base, chunk)], abuf)
      pltpu.sync_copy(abuf, sh_e.at[row_v], add=True)
      pltpu.sync_copy(ones_v, sh_c.at[row_v], add=True)
      return carry

    lax.fori_loop(0, n_chunks, body, 0)

    plsc.subcore_barrier()
    pltpu.sync_copy(sh_s.at[pl.ds(r0, rows_per_tile)],
                    s_out.at[cid, pl.ds(r0, rows_per_tile)])
    pltpu.sync_copy(sh_e.at[pl.ds(r0, rows_per_tile)],
                    e_out.at[cid, pl.ds(r0, rows_per_tile)])
    pltpu.sync_copy(sh_c.at[pl.ds(r0, rows_per_tile)],
                    c_out.at[cid, pl.ds(r0, rows_per_tile)])

  return k(x, row, col, edge_attr, zs, ze, ones)


def _tc_finish(s2, e2, c2, w, b):
  nc, n, d = s2.shape
  da = e2.shape[2]
  blk = 1264
  grid = n // blk
  b2 = b.reshape(1, d)

  def body(s_ref, e_ref, c_ref, w_ref, b_ref, o_ref):
    s = s_ref[0] + s_ref[1]
    e = e_ref[0] + e_ref[1]
    cnt = c_ref[0, :, 0:1] + c_ref[1, :, 0:1]
    acc = jnp.dot(s, w_ref[0:d, :], preferred_element_type=jnp.float32)
    acc = acc + jnp.dot(e, w_ref[d:, :], preferred_element_type=jnp.float32)
    o_ref[...] = acc / jnp.maximum(cnt, 1.0) + b_ref[...]

  return pl.pallas_call(
      body,
      grid=(grid,),
      in_specs=[
          pl.BlockSpec((nc, blk, d), lambda i: (0, i, 0)),
          pl.BlockSpec((nc, blk, da), lambda i: (0, i, 0)),
          pl.BlockSpec((nc, blk, da), lambda i: (0, i, 0)),
          pl.BlockSpec((d + da, d), lambda i: (0, 0)),
          pl.BlockSpec((1, d), lambda i: (0, 0)),
      ],
      out_specs=pl.BlockSpec((blk, d), lambda i: (i, 0)),
      out_shape=jax.ShapeDtypeStruct((n, d), jnp.float32),
  )(s2, e2, c2, w, b2)


def kernel(x, edge_index, edge_attr, W, b):
  row = edge_index[0]
  col = edge_index[1]
  s2, e2, c2 = _sc_segment_sums(x, row, col, edge_attr)
  return _tc_finish(s2, e2, c2, W, b)[: x.shape[0]]
